# baseline (speedup 1.0000x reference)
import jax
import jax.numpy as jnp
from jax.experimental import pallas as pl
from jax.experimental.pallas import tpu as pltpu

_BLK = 32768
_CHK = 512
_EPS = 1e-5


def _rowsum_rep(h):
    t = h + pltpu.roll(h, 4, 0)
    t = t + pltpu.roll(t, 2, 0)
    return t + pltpu.roll(t, 1, 0)


def _ln_mod(h, mask6, red, g, bb):
    mu = _rowsum_rep(h) * (1.0 / 6.0)
    d = (h - mu) * mask6
    var = jnp.dot(red, d * d, preferred_element_type=jnp.float32)
    hn = d * jax.lax.rsqrt(var + _EPS)
    return g * hn + bb


def _body(xyT_ref, tab_ref, prm_ref, whm_ref, wout_ref, red_ref, out_ref,
          itp_ref):
    prm = prm_ref[...]
    w_in_x = prm[:, 0:1]
    w_in_y = prm[:, 1:2]
    b_in = prm[:, 2:3]
    mask6 = prm[:, 3:4]
    tab = tab_ref[...]
    red = red_ref[...]
    iot = jax.lax.broadcasted_iota(jnp.int32, (16, _CHK), 0).astype(jnp.float32)

    for j in range(_BLK // _CHK):
        lo, hi = j * _CHK, (j + 1) * _CHK
        xy = xyT_ref[:, lo:hi]
        cx = jnp.clip((xy[0:1, :] + 1.0) * 7.5, 0.0, 15.0)
        cy = jnp.clip((xy[1:2, :] + 1.0) * 7.5, 0.0, 15.0)
        ohx = jnp.maximum(0.0, 1.0 - jnp.abs(cx - iot))
        ohy = jnp.maximum(0.0, 1.0 - jnp.abs(cy - iot))
        rows = []
        for yv in range(16):
            b = jnp.broadcast_to(ohy[yv:yv + 1, :], (8, _CHK))
            rows.append(b)
            rows.append(b)
        ohy_exp = jnp.concatenate(rows, axis=0)
        w_interp = ohy_exp * jnp.tile(ohx, (16, 1))
        itp_ref[:, lo:hi] = jnp.dot(tab, w_interp,
                                    preferred_element_type=jnp.float32)

    half = _BLK // 2
    for p in range(2):
        lo, hi = p * half, (p + 1) * half
        x = xyT_ref[0:1, lo:hi]
        y = xyT_ref[1:2, lo:hi]
        itp = itp_ref[:, lo:hi]
        g0 = itp[0:8, :]
        bb0 = itp[8:16, :]
        g1 = itp[16:24, :]
        bb1 = itp[24:32, :]
        h = x * w_in_x + y * w_in_y + b_in
        h = h * jax.nn.sigmoid(h)
        h = _ln_mod(h, mask6, red, g0, bb0)
        z = jnp.dot(whm_ref[...], h, preferred_element_type=jnp.float32)
        h = z * jax.nn.sigmoid(z)
        h = _ln_mod(h, mask6, red, g1, bb1)
        o = jnp.dot(wout_ref[...], h, preferred_element_type=jnp.float32)
        out_ref[:, lo:hi] = o[0:3, :]


def kernel(xy, gamma, beta, w_in, b_in, w_h, b_h, w_out, b_out, ln_w, ln_b):
    n = xy.shape[0]
    xyT = xy.T
    g6 = gamma.reshape(2, 6, 256)
    b6 = beta.reshape(2, 6, 256)
    ones_row = jnp.ones((1, 256), jnp.float32)
    t = jnp.zeros((32, 256), jnp.float32)
    t = t.at[0:6].set(ln_w[0][:, None] * g6[0])
    t = t.at[8:14].set(ln_b[0][:, None] * g6[0] + b6[0])
    t = t.at[14:15].set(ones_row)
    t = t.at[16:22].set(ln_w[1][:, None] * g6[1])
    t = t.at[24:30].set(ln_b[1][:, None] * g6[1] + b6[1])
    t = t.at[30:31].set(ones_row)
    tab = t

    def col(v):
        return jnp.pad(v.astype(jnp.float32), (0, 8 - v.shape[0]))

    mask6 = jnp.array([1, 1, 1, 1, 1, 1, 0, 0], jnp.float32)
    prm = jnp.stack([
        col(w_in[:, 0]), col(w_in[:, 1]), col(b_in), mask6,
    ] + [jnp.zeros(8, jnp.float32)] * 12, axis=1)
    whm = jnp.zeros((8, 8), jnp.float32).at[0:6, 0:6].set(w_h[0])
    whm = whm.at[0:6, 6].set(b_h[0])
    wout = jnp.zeros((8, 8), jnp.float32).at[0:3, 0:6].set(w_out)
    wout = wout.at[0:3, 6].set(b_out)
    red = jnp.broadcast_to((mask6 * (1.0 / 6.0))[None, :], (8, 8))

    outT = pl.pallas_call(
        _body,
        out_shape=jax.ShapeDtypeStruct((3, n), jnp.float32),
        grid=(n // _BLK,),
        in_specs=[
            pl.BlockSpec((2, _BLK), lambda i: (0, i)),
            pl.BlockSpec((32, 256), lambda i: (0, 0)),
            pl.BlockSpec((8, 16), lambda i: (0, 0)),
            pl.BlockSpec((8, 8), lambda i: (0, 0)),
            pl.BlockSpec((8, 8), lambda i: (0, 0)),
            pl.BlockSpec((8, 8), lambda i: (0, 0)),
        ],
        out_specs=pl.BlockSpec((3, _BLK), lambda i: (0, i)),
        scratch_shapes=[pltpu.VMEM((32, _BLK), jnp.float32)],
        compiler_params=pltpu.CompilerParams(
            dimension_semantics=("arbitrary",),
        ),
        name="camfield_fused",
    )(xyT, tab, prm, whm, wout, red)
    return outT.T


# speedup vs baseline: 1.0281x; 1.0281x over previous
import jax
import jax.numpy as jnp
from jax.experimental import pallas as pl
from jax.experimental.pallas import tpu as pltpu

_BLK = 65536
_CHK = 512
_EPS = 1e-5


def _rowsum_rep(h):
    t = h + pltpu.roll(h, 4, 0)
    t = t + pltpu.roll(t, 2, 0)
    return t + pltpu.roll(t, 1, 0)


def _ln_mod(h, mask6, red, g, bb):
    mu = _rowsum_rep(h) * (1.0 / 6.0)
    d = (h - mu) * mask6
    var = jnp.dot(red, d * d, preferred_element_type=jnp.float32)
    hn = d * jax.lax.rsqrt(var + _EPS)
    return g * hn + bb


def _body(xyT_ref, tab_ref, prm_ref, whm_ref, wout_ref, red_ref, out_ref,
          itp_ref):
    prm = prm_ref[...]
    w_in_x = prm[:, 0:1]
    w_in_y = prm[:, 1:2]
    b_in = prm[:, 2:3]
    mask6 = prm[:, 3:4]
    tab = tab_ref[...]
    red = red_ref[...]
    iot = jax.lax.broadcasted_iota(jnp.int32, (16, _CHK), 0).astype(jnp.float32)

    for j in range(_BLK // _CHK):
        lo, hi = j * _CHK, (j + 1) * _CHK
        xy = xyT_ref[:, lo:hi]
        cx = jnp.clip((xy[0:1, :] + 1.0) * 7.5, 0.0, 15.0)
        cy = jnp.clip((xy[1:2, :] + 1.0) * 7.5, 0.0, 15.0)
        ohx = jnp.maximum(0.0, 1.0 - jnp.abs(cx - iot))
        ohy = jnp.maximum(0.0, 1.0 - jnp.abs(cy - iot))
        rows = []
        for yv in range(16):
            b = jnp.broadcast_to(ohy[yv:yv + 1, :], (8, _CHK))
            rows.append(b)
            rows.append(b)
        ohy_exp = jnp.concatenate(rows, axis=0)
        w_interp = ohy_exp * jnp.tile(ohx, (16, 1))
        itp_ref[:, lo:hi] = jnp.dot(tab, w_interp,
                                    preferred_element_type=jnp.float32)

    half = _BLK // 2
    for p in range(2):
        lo, hi = p * half, (p + 1) * half
        x = xyT_ref[0:1, lo:hi]
        y = xyT_ref[1:2, lo:hi]
        itp = itp_ref[:, lo:hi]
        g0 = itp[0:8, :]
        bb0 = itp[8:16, :]
        g1 = itp[16:24, :]
        bb1 = itp[24:32, :]
        h = x * w_in_x + y * w_in_y + b_in
        h = h * jax.nn.sigmoid(h)
        h = _ln_mod(h, mask6, red, g0, bb0)
        z = jnp.dot(whm_ref[...], h, preferred_element_type=jnp.float32)
        h = z * jax.nn.sigmoid(z)
        h = _ln_mod(h, mask6, red, g1, bb1)
        o = jnp.dot(wout_ref[...], h, preferred_element_type=jnp.float32)
        out_ref[:, lo:hi] = o[0:3, :]


def kernel(xy, gamma, beta, w_in, b_in, w_h, b_h, w_out, b_out, ln_w, ln_b):
    n = xy.shape[0]
    xyT = xy.T
    g6 = gamma.reshape(2, 6, 256)
    b6 = beta.reshape(2, 6, 256)
    ones_row = jnp.ones((1, 256), jnp.float32)
    t = jnp.zeros((32, 256), jnp.float32)
    t = t.at[0:6].set(ln_w[0][:, None] * g6[0])
    t = t.at[8:14].set(ln_b[0][:, None] * g6[0] + b6[0])
    t = t.at[14:15].set(ones_row)
    t = t.at[16:22].set(ln_w[1][:, None] * g6[1])
    t = t.at[24:30].set(ln_b[1][:, None] * g6[1] + b6[1])
    t = t.at[30:31].set(ones_row)
    tab = t

    def col(v):
        return jnp.pad(v.astype(jnp.float32), (0, 8 - v.shape[0]))

    mask6 = jnp.array([1, 1, 1, 1, 1, 1, 0, 0], jnp.float32)
    prm = jnp.stack([
        col(w_in[:, 0]), col(w_in[:, 1]), col(b_in), mask6,
    ] + [jnp.zeros(8, jnp.float32)] * 12, axis=1)
    whm = jnp.zeros((8, 8), jnp.float32).at[0:6, 0:6].set(w_h[0])
    whm = whm.at[0:6, 6].set(b_h[0])
    wout = jnp.zeros((8, 8), jnp.float32).at[0:3, 0:6].set(w_out)
    wout = wout.at[0:3, 6].set(b_out)
    red = jnp.broadcast_to((mask6 * (1.0 / 6.0))[None, :], (8, 8))

    outT = pl.pallas_call(
        _body,
        out_shape=jax.ShapeDtypeStruct((3, n), jnp.float32),
        grid=(n // _BLK,),
        in_specs=[
            pl.BlockSpec((2, _BLK), lambda i: (0, i)),
            pl.BlockSpec((32, 256), lambda i: (0, 0)),
            pl.BlockSpec((8, 16), lambda i: (0, 0)),
            pl.BlockSpec((8, 8), lambda i: (0, 0)),
            pl.BlockSpec((8, 8), lambda i: (0, 0)),
            pl.BlockSpec((8, 8), lambda i: (0, 0)),
        ],
        out_specs=pl.BlockSpec((3, _BLK), lambda i: (0, i)),
        scratch_shapes=[pltpu.VMEM((32, _BLK), jnp.float32)],
        compiler_params=pltpu.CompilerParams(
            dimension_semantics=("arbitrary",),
        ),
        name="camfield_fused",
    )(xyT, tab, prm, whm, wout, red)
    return outT.T
